# baseline (speedup 1.0000x reference)
import jax
import jax.numpy as jnp
from jax.experimental import pallas as pl

B = 16384
D = 1024
BLK = 256


def _tc_body(idx_ref, tab_ref, o_ref):
    idxb = idx_ref[0]
    w0 = tab_ref[pl.ds(0, 1), :]
    w1 = tab_ref[pl.ds(1, 1), :]
    o_ref[...] = jnp.where(idxb == 0, w0, w1)


def kernel(domain_idx, embed_weight):
    idx3 = domain_idx.astype(jnp.int32).reshape(B // BLK, BLK, 1)
    return pl.pallas_call(
        _tc_body,
        out_shape=jax.ShapeDtypeStruct((B, D), jnp.float32),
        grid=(B // BLK,),
        in_specs=[
            pl.BlockSpec((1, BLK, 1), lambda i: (i, 0, 0)),
            pl.BlockSpec((2, D), lambda i: (0, 0)),
        ],
        out_specs=pl.BlockSpec((BLK, D), lambda i: (i, 0)),
    )(idx3, embed_weight)
